# Initial kernel scaffold; baseline (speedup 1.0000x reference)
import jax
import jax.numpy as jnp
from jax.experimental import pallas as pl


def kernel(x, edge_index, W1_0, W1_1, b1, W2_0, W2_1, b2, Wl, bl):
    raise NotImplementedError("write your pallas kernel here")


# speedup vs baseline: 65.4830x; 65.4830x over previous
import dataclasses
import functools

import jax
import jax.numpy as jnp
from jax import lax
from jax.experimental import pallas as pl
from jax.experimental.pallas import tpu as pltpu
from jax.experimental.pallas import tpu_sc as plsc

N = 10000
E = 320000
NP = 10240
NC = 2
NS = 16
NW = NC * NS
EPW = E // NW
L = 16

_mesh = plsc.VectorSubcoreMesh(core_axis_name="c", subcore_axis_name="s")

_sc_params = pltpu.CompilerParams()
if "needs_layout_passes" in pltpu.CompilerParams.__dataclass_fields__:
    _sc_params = dataclasses.replace(_sc_params, needs_layout_passes=False)


@functools.partial(
    pl.kernel,
    out_type=jax.ShapeDtypeStruct((NW, NP), jnp.float32),
    mesh=_mesh,
    compiler_params=_sc_params,
    scratch_types=[
        pltpu.VMEM((EPW,), jnp.int32),
        pltpu.VMEM((EPW,), jnp.int32),
        pltpu.VMEM((NP,), jnp.float32),
    ],
)
def _sc_deg(row_hbm, col_hbm, zeros_hbm, out_hbm, row_v, col_v, acc_v):
    wid = lax.axis_index("c") * NS + lax.axis_index("s")
    base = wid * EPW
    pltpu.sync_copy(row_hbm.at[pl.ds(base, EPW)], row_v)
    pltpu.sync_copy(col_hbm.at[pl.ds(base, EPW)], col_v)
    pltpu.sync_copy(zeros_hbm.at[0], acc_v)
    ones = jnp.ones((L,), jnp.float32)

    @pl.loop(0, EPW, step=L)
    def _(i):
        r = row_v[pl.ds(i, L)]
        c = col_v[pl.ds(i, L)]
        plsc.addupdate_scatter(acc_v, [r], ones, mask=r != c)

    pltpu.sync_copy(acc_v, out_hbm.at[wid])


@functools.partial(
    pl.kernel,
    out_type=jax.ShapeDtypeStruct((NW, 3, NP), jnp.float32),
    mesh=_mesh,
    compiler_params=_sc_params,
    scratch_types=[
        pltpu.VMEM((EPW,), jnp.int32),
        pltpu.VMEM((EPW,), jnp.int32),
        pltpu.VMEM((3, NP), jnp.float32),
        pltpu.VMEM((3, NP), jnp.float32),
    ],
)
def _sc_prop(row_hbm, col_hbm, z_hbm, zeros_hbm, out_hbm, row_v, col_v, z_v,
             acc_v):
    wid = lax.axis_index("c") * NS + lax.axis_index("s")
    base = wid * EPW
    pltpu.sync_copy(row_hbm.at[pl.ds(base, EPW)], row_v)
    pltpu.sync_copy(col_hbm.at[pl.ds(base, EPW)], col_v)
    pltpu.sync_copy(z_hbm, z_v)
    pltpu.sync_copy(zeros_hbm, acc_v)

    @pl.loop(0, EPW, step=L)
    def _(i):
        r = row_v[pl.ds(i, L)]
        c = col_v[pl.ds(i, L)]
        m = r != c
        for f in range(3):
            fs = jnp.full((L,), f, jnp.int32)
            v = plsc.load_gather(z_v, [fs, r])
            plsc.addupdate_scatter(acc_v, [fs, c], v, mask=m)

    pltpu.sync_copy(acc_v, out_hbm.at[wid])


def _tc_mm1(xp, Wcat):
    def body(x_ref, w_ref, o_ref):
        o_ref[...] = lax.dot_general(
            w_ref[...], x_ref[...], (((0,), (1,)), ((), ())),
            preferred_element_type=jnp.float32)

    return pl.pallas_call(
        body, out_shape=jax.ShapeDtypeStruct((6, NP), jnp.float32))(xp, Wcat)


def _tc_z1(degp, yT):
    def body(degp_ref, yT_ref, z_ref, dinv_ref):
        deg = jnp.sum(degp_ref[...], axis=0, keepdims=True)
        dinv = jnp.where(deg > 0, lax.rsqrt(deg), 0.0)
        z_ref[...] = dinv * yT_ref[3:6, :]
        dinv_ref[...] = dinv

    return pl.pallas_call(
        body,
        out_shape=(jax.ShapeDtypeStruct((3, NP), jnp.float32),
                   jax.ShapeDtypeStruct((1, NP), jnp.float32)))(degp, yT)


def _tc_h1(s1p, yT, dinvT, b1c):
    def body(s1p_ref, yT_ref, dinv_ref, b1_ref, h1_ref, z2_ref):
        s1 = jnp.sum(s1p_ref[...], axis=0)
        dinv = dinv_ref[...]
        p1 = -dinv * s1
        h1 = jnp.maximum(yT_ref[0:3, :] + p1 + b1_ref[...], 0.0)
        h1_ref[...] = h1
        z2_ref[...] = dinv * h1

    return pl.pallas_call(
        body,
        out_shape=(jax.ShapeDtypeStruct((3, NP), jnp.float32),
                   jax.ShapeDtypeStruct((3, NP), jnp.float32)))(
            s1p, yT, dinvT, b1c)


def _tc_final(s2p, h1T, dinvT, W2t0, W2t1, b2c, Wl, blr):
    def body(s2p_ref, h1_ref, dinv_ref, w20_ref, w21_ref, b2_ref, wl_ref,
             bl_ref, o_ref):
        s2 = jnp.sum(s2p_ref[...], axis=0)
        p2 = -dinv_ref[...] * s2
        h2 = lax.dot_general(w20_ref[...], h1_ref[...],
                             (((1,), (0,)), ((), ())),
                             preferred_element_type=jnp.float32)
        h2 = h2 + lax.dot_general(w21_ref[...], p2,
                                  (((1,), (0,)), ((), ())),
                                  preferred_element_type=jnp.float32)
        h2 = jnp.maximum(h2 + b2_ref[...], 0.0)
        out = lax.dot_general(h2, wl_ref[...], (((0,), (0,)), ((), ())),
                              preferred_element_type=jnp.float32)
        o_ref[...] = out + bl_ref[...]

    return pl.pallas_call(
        body, out_shape=jax.ShapeDtypeStruct((NP, 128), jnp.float32))(
            s2p, h1T, dinvT, W2t0, W2t1, b2c, Wl, blr)


def kernel(x, edge_index, W1_0, W1_1, b1, W2_0, W2_1, b2, Wl, bl):
    xp = jnp.pad(x, ((0, NP - N), (0, 0)))
    row = edge_index[0]
    col = edge_index[1]
    Wcat = jnp.concatenate([W1_0, W1_1], axis=1)
    b1c = b1.reshape(3, 1)
    W2t0 = W2_0.T
    W2t1 = W2_1.T
    b2c = b2.reshape(128, 1)
    blr = bl.reshape(1, 128)
    zeros3 = jnp.zeros((3, NP), jnp.float32)

    yT = _tc_mm1(xp, Wcat)
    degp = _sc_deg(row, col, zeros3)
    z1T, dinvT = _tc_z1(degp, yT)
    s1p = _sc_prop(row, col, z1T, zeros3)
    h1T, z2T = _tc_h1(s1p, yT, dinvT, b1c)
    s2p = _sc_prop(row, col, z2T, zeros3)
    out = _tc_final(s2p, h1T, dinvT, W2t0, W2t1, b2c, Wl, blr)
    return out[:N]


# speedup vs baseline: 67.4146x; 1.0295x over previous
import dataclasses
import functools

import jax
import jax.numpy as jnp
from jax import lax
from jax.experimental import pallas as pl
from jax.experimental.pallas import tpu as pltpu
from jax.experimental.pallas import tpu_sc as plsc

N = 10000
E = 320000
NP = 10240
NC = 2
NS = 16
NW = NC * NS
EPW = E // NW
L = 16

_mesh = plsc.VectorSubcoreMesh(core_axis_name="c", subcore_axis_name="s")

_sc_params = pltpu.CompilerParams()
if "needs_layout_passes" in pltpu.CompilerParams.__dataclass_fields__:
    _sc_params = dataclasses.replace(_sc_params, needs_layout_passes=False)


@functools.partial(
    pl.kernel,
    out_type=jax.ShapeDtypeStruct((NW, NP), jnp.float32),
    mesh=_mesh,
    compiler_params=_sc_params,
    scratch_types=[
        pltpu.VMEM((EPW,), jnp.int32),
        pltpu.VMEM((EPW,), jnp.int32),
        pltpu.VMEM((NP,), jnp.float32),
        pltpu.SemaphoreType.DMA,
    ],
)
def _sc_deg(row_hbm, col_hbm, zeros_hbm, out_hbm, row_v, col_v, acc_v, sem):
    wid = lax.axis_index("c") * NS + lax.axis_index("s")
    base = wid * EPW
    c1 = pltpu.async_copy(row_hbm.at[pl.ds(base, EPW)], row_v, sem)
    c2 = pltpu.async_copy(col_hbm.at[pl.ds(base, EPW)], col_v, sem)
    c3 = pltpu.async_copy(zeros_hbm.at[0], acc_v, sem)
    c1.wait()
    c2.wait()
    c3.wait()
    ones = jnp.ones((L,), jnp.float32)

    @pl.loop(0, EPW, step=L * 5)
    def _(i):
        for u in range(5):
            r = row_v[pl.ds(i + u * L, L)]
            c = col_v[pl.ds(i + u * L, L)]
            plsc.addupdate_scatter(acc_v, [r], ones, mask=r != c)

    pltpu.sync_copy(acc_v, out_hbm.at[wid])


@functools.partial(
    pl.kernel,
    out_type=jax.ShapeDtypeStruct((NW, 3, NP), jnp.float32),
    mesh=_mesh,
    compiler_params=_sc_params,
    scratch_types=[
        pltpu.VMEM((EPW,), jnp.int32),
        pltpu.VMEM((EPW,), jnp.int32),
        pltpu.VMEM((3, NP), jnp.float32),
        pltpu.VMEM((3, NP), jnp.float32),
        pltpu.SemaphoreType.DMA,
    ],
)
def _sc_prop(row_hbm, col_hbm, z_hbm, zeros_hbm, out_hbm, row_v, col_v, z_v,
             acc_v, sem):
    wid = lax.axis_index("c") * NS + lax.axis_index("s")
    base = wid * EPW
    c1 = pltpu.async_copy(row_hbm.at[pl.ds(base, EPW)], row_v, sem)
    c2 = pltpu.async_copy(col_hbm.at[pl.ds(base, EPW)], col_v, sem)
    c3 = pltpu.async_copy(z_hbm, z_v, sem)
    c4 = pltpu.async_copy(zeros_hbm, acc_v, sem)
    c1.wait()
    c2.wait()
    c3.wait()
    c4.wait()

    @pl.loop(0, EPW, step=L * 5)
    def _(i):
        for u in range(5):
            r = row_v[pl.ds(i + u * L, L)]
            c = col_v[pl.ds(i + u * L, L)]
            m = r != c
            for f in range(3):
                fs = jnp.full((L,), f, jnp.int32)
                v = plsc.load_gather(z_v, [fs, r])
                plsc.addupdate_scatter(acc_v, [fs, c], v, mask=m)

    pltpu.sync_copy(acc_v, out_hbm.at[wid])


def _tc_mm1(xp, Wcat):
    def body(x_ref, w_ref, o_ref):
        o_ref[...] = lax.dot_general(
            w_ref[...], x_ref[...], (((0,), (1,)), ((), ())),
            preferred_element_type=jnp.float32)

    return pl.pallas_call(
        body, out_shape=jax.ShapeDtypeStruct((6, NP), jnp.float32))(xp, Wcat)


def _tc_z1(degp, yT):
    def body(degp_ref, yT_ref, z_ref, dinv_ref):
        deg = jnp.sum(degp_ref[...], axis=0, keepdims=True)
        dinv = jnp.where(deg > 0, lax.rsqrt(deg), 0.0)
        z_ref[...] = dinv * yT_ref[3:6, :]
        dinv_ref[...] = dinv

    return pl.pallas_call(
        body,
        out_shape=(jax.ShapeDtypeStruct((3, NP), jnp.float32),
                   jax.ShapeDtypeStruct((1, NP), jnp.float32)))(degp, yT)


def _tc_h1(s1p, yT, dinvT, b1c):
    def body(s1p_ref, yT_ref, dinv_ref, b1_ref, h1_ref, z2_ref):
        s1 = jnp.sum(s1p_ref[...], axis=0)
        dinv = dinv_ref[...]
        p1 = -dinv * s1
        h1 = jnp.maximum(yT_ref[0:3, :] + p1 + b1_ref[...], 0.0)
        h1_ref[...] = h1
        z2_ref[...] = dinv * h1

    return pl.pallas_call(
        body,
        out_shape=(jax.ShapeDtypeStruct((3, NP), jnp.float32),
                   jax.ShapeDtypeStruct((3, NP), jnp.float32)))(
            s1p, yT, dinvT, b1c)


def _tc_final(s2p, h1T, dinvT, W2t0, W2t1, b2c, Wl, blr):
    def body(s2p_ref, h1_ref, dinv_ref, w20_ref, w21_ref, b2_ref, wl_ref,
             bl_ref, o_ref):
        s2 = jnp.sum(s2p_ref[...], axis=0)
        p2 = -dinv_ref[...] * s2
        h2 = lax.dot_general(w20_ref[...], h1_ref[...],
                             (((1,), (0,)), ((), ())),
                             preferred_element_type=jnp.float32)
        h2 = h2 + lax.dot_general(w21_ref[...], p2,
                                  (((1,), (0,)), ((), ())),
                                  preferred_element_type=jnp.float32)
        h2 = jnp.maximum(h2 + b2_ref[...], 0.0)
        out = lax.dot_general(h2, wl_ref[...], (((0,), (0,)), ((), ())),
                              preferred_element_type=jnp.float32)
        o_ref[...] = out + bl_ref[...]

    return pl.pallas_call(
        body, out_shape=jax.ShapeDtypeStruct((NP, 128), jnp.float32))(
            s2p, h1T, dinvT, W2t0, W2t1, b2c, Wl, blr)


def kernel(x, edge_index, W1_0, W1_1, b1, W2_0, W2_1, b2, Wl, bl):
    xp = jnp.pad(x, ((0, NP - N), (0, 0)))
    row = edge_index[0]
    col = edge_index[1]
    Wcat = jnp.concatenate([W1_0, W1_1], axis=1)
    b1c = b1.reshape(3, 1)
    W2t0 = W2_0.T
    W2t1 = W2_1.T
    b2c = b2.reshape(128, 1)
    blr = bl.reshape(1, 128)
    zeros3 = jnp.zeros((3, NP), jnp.float32)

    yT = _tc_mm1(xp, Wcat)
    degp = _sc_deg(row, col, zeros3)
    z1T, dinvT = _tc_z1(degp, yT)
    s1p = _sc_prop(row, col, z1T, zeros3)
    h1T, z2T = _tc_h1(s1p, yT, dinvT, b1c)
    s2p = _sc_prop(row, col, z2T, zeros3)
    out = _tc_final(s2p, h1T, dinvT, W2t0, W2t1, b2c, Wl, blr)
    return out[:N]
